# baseline (speedup 1.0000x reference)
import jax
import jax.numpy as jnp
from jax.experimental import pallas as pl
from jax.experimental.pallas import tpu as pltpu

D_MODEL = 2048
N_EXPERTS = 16
CAPACITY_FACTOR = 1.0
EPSILON = 1e-06
S_BLK = 256


def _gate_kernel(x_ref, w_ref, b_ref, o_ref, logits_ref):
    B = x_ref.shape[0]
    i = pl.program_id(0)
    n = pl.num_programs(0)
    sl = jax.lax.rem(i, 2)

    @pl.when(i < n - 1)
    def _dot_phase():
        w = w_ref[...]
        bias = b_ref[...]
        for b in range(B):
            logits_ref[sl, b] = (
                jnp.dot(x_ref[b], w, preferred_element_type=jnp.float32) + bias)

    @pl.when(i > 0)
    def _routing_phase():
        masked = []
        for b in range(B):
            logits = logits_ref[1 - sl, b]

            m1 = jnp.max(logits, axis=-1, keepdims=True)
            m2 = jnp.max(jnp.where(logits == m1, -jnp.inf, logits),
                         axis=-1, keepdims=True)
            hot = logits >= m2

            e = jnp.exp(logits - m1)
            rowsum = jnp.sum(e, axis=-1, keepdims=True)
            masked.append(jnp.where(hot, e / rowsum, 0.0))

        denom = masked[0]
        for b in range(1, B):
            denom = denom + masked[b]
        scale = CAPACITY_FACTOR * B / (denom + EPSILON)
        for b in range(B):
            o_ref[b] = masked[b] * scale


def kernel(X, W, b):
    B, S, D = X.shape
    nblk = S // S_BLK
    return pl.pallas_call(
        _gate_kernel,
        grid=(nblk + 1,),
        in_specs=[
            pl.BlockSpec((B, S_BLK, D), lambda i: (0, jnp.minimum(i, nblk - 1), 0)),
            pl.BlockSpec((D, N_EXPERTS), lambda i: (0, 0)),
            pl.BlockSpec((1, N_EXPERTS), lambda i: (0, 0)),
        ],
        out_specs=pl.BlockSpec((B, S_BLK, N_EXPERTS),
                               lambda i: (0, jnp.maximum(i - 1, 0), 0)),
        out_shape=jax.ShapeDtypeStruct((B, S, N_EXPERTS), jnp.float32),
        scratch_shapes=[pltpu.VMEM((2, B, S_BLK, N_EXPERTS), jnp.float32)],
    )(X, W, b.reshape(1, N_EXPERTS))


# speedup vs baseline: 1.0529x; 1.0529x over previous
import jax
import jax.numpy as jnp
from jax.experimental import pallas as pl

D_MODEL = 2048
N_EXPERTS = 16
CAPACITY_FACTOR = 1.0
EPSILON = 1e-06
S_BLK = 256


def _gate_kernel(x_ref, w_ref, b_ref, o_ref):
    B, S, D = x_ref.shape
    w = w_ref[...]
    bias = b_ref[...]

    masked = []
    for b in range(B):
        logits = jnp.dot(x_ref[b], w, preferred_element_type=jnp.float32) + bias

        m1 = jnp.max(logits, axis=-1, keepdims=True)
        m2 = jnp.max(jnp.where(logits == m1, -jnp.inf, logits), axis=-1, keepdims=True)
        hot = logits >= m2

        e = jnp.exp(logits - m1)
        rinv = pl.reciprocal(jnp.sum(e, axis=-1, keepdims=True), approx=False)
        masked.append(jnp.where(hot, e, 0.0) * rinv)

    denom = masked[0]
    for b in range(1, B):
        denom = denom + masked[b]
    inv_cap = 1.0 / (CAPACITY_FACTOR * B)
    scale = pl.reciprocal((denom + EPSILON) * inv_cap, approx=False)
    for b in range(B):
        o_ref[b] = masked[b] * scale


def kernel(X, W, b):
    B, S, D = X.shape
    return pl.pallas_call(
        _gate_kernel,
        grid=(S // S_BLK,),
        in_specs=[
            pl.BlockSpec((B, S_BLK, D), lambda i: (0, i, 0)),
            pl.BlockSpec((D, N_EXPERTS), lambda i: (0, 0)),
            pl.BlockSpec((1, N_EXPERTS), lambda i: (0, 0)),
        ],
        out_specs=pl.BlockSpec((B, S_BLK, N_EXPERTS), lambda i: (0, i, 0)),
        out_shape=jax.ShapeDtypeStruct((B, S, N_EXPERTS), jnp.float32),
    )(X, W, b.reshape(1, N_EXPERTS))
